# baseline (speedup 1.0000x reference)
import jax
import jax.numpy as jnp
from jax.experimental import pallas as pl
from jax.experimental.pallas import tpu as pltpu

_BL = 256
_BO = 128


def _body(tok_ref, en_ref, w_ref, eb_ref, pe_ref, out_ref):
    j = pl.program_id(0)
    h = jax.lax.rem(j, 2) * _BO
    e = jax.lax.dot_general(
        w_ref[:], en_ref[:, pl.ds(j * _BO, _BO)], (((0,), (0,)), ((), ())),
        preferred_element_type=jnp.float32)
    out_ref[0, :, :] = e + eb_ref[:]
    out_ref[1:, :, :] = tok_ref[:, :, pl.ds(h, _BO)] + pe_ref[:]


def kernel(tokens, energies, W, b, emb):
    batch, n_in, tsz = tokens.shape
    n_tok = emb.shape[0]
    tokens_t = tokens.transpose(1, 2, 0)
    energies_t = energies.T
    pe = emb[1:].reshape(n_in, tsz, 1)
    eb = (b + emb[0]).reshape(tsz, 1)

    grid = (batch // _BO,)
    resident = pl.BlockSpec(memory_space=pltpu.MemorySpace.VMEM)
    out_t = pl.pallas_call(
        _body,
        grid=grid,
        in_specs=[
            pl.BlockSpec((n_in, tsz, _BL), lambda j: (0, 0, j // 2)),
            resident,
            resident,
            resident,
            resident,
        ],
        out_specs=pl.BlockSpec((n_tok, tsz, _BO), lambda j: (0, 0, j)),
        out_shape=jax.ShapeDtypeStruct((n_tok, tsz, batch), jnp.float32),
    )(tokens_t, energies_t, W, eb, pe)
    return out_t.transpose(2, 0, 1)
